# Initial kernel scaffold; baseline (speedup 1.0000x reference)
import functools

import jax
import jax.numpy as jnp
from jax import lax
from jax.experimental import pallas as pl
from jax.experimental.pallas import tpu as pltpu
from jax.experimental.pallas import tpu_sc as plsc

NC = 2
NS = 16
NW = NC * NS
LANES = 16


@functools.lru_cache(maxsize=None)
def _make_pool_kernel(B, L, D):
    EPG = 2
    GL = EPG * L
    RPW = B // NW
    STEPS = RPW // EPG
    NREG = D // LANES

    mesh = plsc.VectorSubcoreMesh(core_axis_name="c", subcore_axis_name="s")

    @functools.partial(
        pl.kernel,
        mesh=mesh,
        out_type=jax.ShapeDtypeStruct((B, D), jnp.float32),
        scratch_types=[
            pltpu.VMEM((STEPS, GL), jnp.int32),
            pltpu.VMEM((GL, D), jnp.float32),
            pltpu.VMEM((RPW, D), jnp.float32),
            pltpu.SemaphoreType.DMA,
        ],
    )
    def pool(x_hbm, table_hbm, out_hbm, idx_v, rows_v, pooled_v, sem):
        cid = lax.axis_index("c")
        sid = lax.axis_index("s")
        wid = sid * NC + cid
        pltpu.sync_copy(x_hbm.at[pl.ds(wid * STEPS, STEPS)], idx_v)

        def step(g, carry):
            pltpu.async_copy(table_hbm.at[idx_v.at[g]], rows_v, sem).wait()
            for e in range(EPG):
                def body(j, accs):
                    r = e * L + j
                    return tuple(accs[k] + rows_v[r, pl.ds(k * LANES, LANES)]
                                 for k in range(NREG))
                accs = tuple(jnp.zeros((LANES,), jnp.float32)
                             for _ in range(NREG))
                accs = lax.fori_loop(0, L, body, accs)
                row = g * EPG + e
                for k in range(NREG):
                    pooled_v[row, pl.ds(k * LANES, LANES)] = accs[k]
            return carry

        lax.fori_loop(0, STEPS, step, 0)
        pltpu.sync_copy(pooled_v, out_hbm.at[pl.ds(wid * RPW, RPW)])

    return pool


@functools.lru_cache(maxsize=None)
def _make_mlp_kernel(B, D, H, O, BT):
    def body(p_ref, w1_ref, b1_ref, w2_ref, b2_ref, o_ref):
        h = jnp.dot(p_ref[...], w1_ref[...],
                    preferred_element_type=jnp.float32) + b1_ref[...]
        h = jnp.maximum(h, 0.0)
        o_ref[...] = jnp.dot(h, w2_ref[...],
                             preferred_element_type=jnp.float32) + b2_ref[...]

    return pl.pallas_call(
        body,
        grid=(B // BT,),
        in_specs=[
            pl.BlockSpec((BT, D), lambda i: (i, 0)),
            pl.BlockSpec((D, H), lambda i: (0, 0)),
            pl.BlockSpec((1, H), lambda i: (0, 0)),
            pl.BlockSpec((H, O), lambda i: (0, 0)),
            pl.BlockSpec((1, O), lambda i: (0, 0)),
        ],
        out_specs=pl.BlockSpec((BT, O), lambda i: (i, 0)),
        out_shape=jax.ShapeDtypeStruct((B, O), jnp.float32),
    )


def kernel(x, table, W1, b1, W2, b2):
    B, L = x.shape
    V, D = table.shape
    H = W1.shape[0]
    O = W2.shape[0]
    x2 = x.reshape(B // 2, 2 * L)
    sums = _make_pool_kernel(B, L, D)(x2, table)
    mlp = _make_mlp_kernel(B, D, H, O, 2048)
    return mlp(sums, W1.T / float(L), b1.reshape(1, H), W2.T, b2.reshape(1, O))


# speedup vs baseline: 2.0948x; 2.0948x over previous
import functools

import jax
import jax.numpy as jnp
from jax import lax
from jax.experimental import pallas as pl
from jax.experimental.pallas import tpu as pltpu
from jax.experimental.pallas import tpu_sc as plsc

NC = 2
NS = 16
NW = NC * NS
LANES = 16


@functools.lru_cache(maxsize=None)
def _make_pool_kernel(B, L, D):
    EPG = 2
    GL = EPG * L
    RPW = B // NW
    STEPS = RPW // EPG
    NREG = D // LANES

    mesh = plsc.VectorSubcoreMesh(core_axis_name="c", subcore_axis_name="s")

    @functools.partial(
        pl.kernel,
        mesh=mesh,
        compiler_params=pltpu.CompilerParams(use_tc_tiling_on_sc=False),
        out_type=jax.ShapeDtypeStruct((B, D), jnp.float32),
        scratch_types=[
            pltpu.VMEM((STEPS, GL), jnp.int32),
            pltpu.VMEM((GL, D), jnp.float32),
            pltpu.VMEM((RPW, D), jnp.float32),
            pltpu.SemaphoreType.DMA,
        ],
    )
    def pool(x_hbm, table_hbm, out_hbm, idx_v, rows_v, pooled_v, sem):
        cid = lax.axis_index("c")
        sid = lax.axis_index("s")
        wid = sid * NC + cid
        pltpu.sync_copy(x_hbm.at[pl.ds(wid * STEPS, STEPS)], idx_v)

        def step(g, carry):
            pltpu.async_copy(table_hbm.at[idx_v.at[g]], rows_v, sem).wait()
            for e in range(EPG):
                def body(j, accs):
                    r = e * L + j
                    return tuple(accs[k] + rows_v[r, pl.ds(k * LANES, LANES)]
                                 for k in range(NREG))
                accs = tuple(jnp.zeros((LANES,), jnp.float32)
                             for _ in range(NREG))
                accs = lax.fori_loop(0, L, body, accs)
                row = g * EPG + e
                for k in range(NREG):
                    pooled_v[row, pl.ds(k * LANES, LANES)] = accs[k]
            return carry

        lax.fori_loop(0, STEPS, step, 0)
        pltpu.sync_copy(pooled_v, out_hbm.at[pl.ds(wid * RPW, RPW)])

    return pool


@functools.lru_cache(maxsize=None)
def _make_mlp_kernel(B, D, H, O, BT):
    def body(p_ref, w1_ref, b1_ref, w2_ref, b2_ref, o_ref):
        h = jnp.dot(p_ref[...], w1_ref[...],
                    preferred_element_type=jnp.float32) + b1_ref[...]
        h = jnp.maximum(h, 0.0)
        o_ref[...] = jnp.dot(h, w2_ref[...],
                             preferred_element_type=jnp.float32) + b2_ref[...]

    return pl.pallas_call(
        body,
        grid=(B // BT,),
        in_specs=[
            pl.BlockSpec((BT, D), lambda i: (i, 0)),
            pl.BlockSpec((D, H), lambda i: (0, 0)),
            pl.BlockSpec((1, H), lambda i: (0, 0)),
            pl.BlockSpec((H, O), lambda i: (0, 0)),
            pl.BlockSpec((1, O), lambda i: (0, 0)),
        ],
        out_specs=pl.BlockSpec((BT, O), lambda i: (i, 0)),
        out_shape=jax.ShapeDtypeStruct((B, O), jnp.float32),
    )


def kernel(x, table, W1, b1, W2, b2):
    B, L = x.shape
    V, D = table.shape
    H = W1.shape[0]
    O = W2.shape[0]
    x2 = x.reshape(B // 2, 2 * L)
    sums = _make_pool_kernel(B, L, D)(x2, table)
    mlp = _make_mlp_kernel(B, D, H, O, 2048)
    return mlp(sums, W1.T / float(L), b1.reshape(1, H), W2.T, b2.reshape(1, O))


# speedup vs baseline: 2.7631x; 1.3190x over previous
import functools

import jax
import jax.numpy as jnp
from jax import lax
from jax.experimental import pallas as pl
from jax.experimental.pallas import tpu as pltpu
from jax.experimental.pallas import tpu_sc as plsc

NC = 2
NS = 16
NW = NC * NS
LANES = 16


@functools.lru_cache(maxsize=None)
def _make_pool_kernel(B, L, D):
    EPG = 1
    GL = EPG * L
    RPW = B // NW
    STEPS = RPW // EPG
    NREG = D // LANES
    NBUF = 8
    OUTER = STEPS // NBUF

    mesh = plsc.VectorSubcoreMesh(core_axis_name="c", subcore_axis_name="s")

    @functools.partial(
        pl.kernel,
        mesh=mesh,
        compiler_params=pltpu.CompilerParams(use_tc_tiling_on_sc=False),
        out_type=jax.ShapeDtypeStruct((B, D), jnp.float32),
        scratch_types=[
            pltpu.VMEM((STEPS, GL), jnp.int32),
            pltpu.VMEM((NBUF, GL, D), jnp.float32),
            pltpu.VMEM((RPW, D), jnp.float32),
        ] + [pltpu.SemaphoreType.DMA] * NBUF,
    )
    def pool(x_hbm, table_hbm, out_hbm, idx_v, rows_v, pooled_v, *sems):
        cid = lax.axis_index("c")
        sid = lax.axis_index("s")
        wid = sid * NC + cid
        pltpu.sync_copy(x_hbm.at[pl.ds(wid * STEPS, STEPS)], idx_v)

        def fire(g, b):
            pltpu.async_copy(table_hbm.at[idx_v.at[g]], rows_v.at[b], sems[b])

        def consume(g, b):
            pltpu.make_async_copy(table_hbm.at[idx_v.at[g]], rows_v.at[b],
                                  sems[b]).wait()
            for e in range(EPG):
                def body(j, accs):
                    out = accs
                    for u in range(2):
                        r = e * L + 2 * j + u
                        out = tuple(out[k] + rows_v[b, r, pl.ds(k * LANES, LANES)]
                                    for k in range(NREG))
                    return out
                accs = tuple(jnp.zeros((LANES,), jnp.float32)
                             for _ in range(NREG))
                accs = lax.fori_loop(0, L // 2, body, accs)
                row = g * EPG + e
                for k in range(NREG):
                    pooled_v[row, pl.ds(k * LANES, LANES)] = accs[k]

        for b in range(NBUF):
            fire(b, b)

        def step(i, carry):
            for b in range(NBUF):
                g = i * NBUF + b
                consume(g, b)
                fire(g + NBUF, b)
            return carry

        lax.fori_loop(0, OUTER - 1, step, 0)
        for b in range(NBUF):
            consume((OUTER - 1) * NBUF + b, b)

        pltpu.sync_copy(pooled_v, out_hbm.at[pl.ds(wid * RPW, RPW)])

    return pool


@functools.lru_cache(maxsize=None)
def _make_mlp_kernel(B, D, H, O, BT):
    def body(p_ref, w1_ref, b1_ref, w2_ref, b2_ref, o_ref):
        h = jnp.dot(p_ref[...], w1_ref[...],
                    preferred_element_type=jnp.float32) + b1_ref[...]
        h = jnp.maximum(h, 0.0)
        o_ref[...] = jnp.dot(h, w2_ref[...],
                             preferred_element_type=jnp.float32) + b2_ref[...]

    return pl.pallas_call(
        body,
        grid=(B // BT,),
        in_specs=[
            pl.BlockSpec((BT, D), lambda i: (i, 0)),
            pl.BlockSpec((D, H), lambda i: (0, 0)),
            pl.BlockSpec((1, H), lambda i: (0, 0)),
            pl.BlockSpec((H, O), lambda i: (0, 0)),
            pl.BlockSpec((1, O), lambda i: (0, 0)),
        ],
        out_specs=pl.BlockSpec((BT, O), lambda i: (i, 0)),
        out_shape=jax.ShapeDtypeStruct((B, O), jnp.float32),
    )


def kernel(x, table, W1, b1, W2, b2):
    B, L = x.shape
    V, D = table.shape
    H = W1.shape[0]
    O = W2.shape[0]
    sums = _make_pool_kernel(B, L, D)(x, table)
    mlp = _make_mlp_kernel(B, D, H, O, 2048)
    return mlp(sums, W1.T / float(L), b1.reshape(1, H), W2.T, b2.reshape(1, O))
